# baseline (speedup 1.0000x reference)
import jax
import jax.numpy as jnp
from jax import lax
from jax.experimental import pallas as pl
from jax.experimental.pallas import tpu as pltpu
from jax.experimental.pallas import tpu_sc as plsc

N_CELLS = 1024
N_GENES = 10000
N_FRAG = 5_000_000

NC = 2
NS = 16

PASSES = 3
CHUNK_CELLS = 171
CHUNK_BINS = CHUNK_CELLS * N_GENES
ZSLICE = 107_008
SP_BINS = ZSLICE * NS
TRASH_BASE = CHUNK_BINS
PAD_VAL = 10_260_000

W = 2048
H = W // 2
NWIN = 156
FT = W * NWIN
NP = FT * NS

RBLK = 128


def _sc_body(ix_hbm, cnt_hbm, bins, idxa, idxb, idxc, oxa1, oxa2, oxb1,
             oxb2, oxc1, oxc2, fill, ones, rowbuf,
             sem_la, sem_lb, sem_lc, sem_sa, sem_sb, sem_sc, sem_z):
    k = lax.axis_index("c")
    s = lax.axis_index("s")

    lane = lax.broadcasted_iota(jnp.int32, (16,), 0)
    trash_vec = TRASH_BASE + s * 16 + lane

    def buf_fill(ref, n, val):
        def body(r, _):
            ref[pl.ds(r * 16, 16)] = jnp.full((16,), val, jnp.float32)
            return 0

        lax.fori_loop(0, n // 16, body, 0)

    buf_fill(fill, W, 0.0)
    buf_fill(ones, H, 1.0)

    def load(idx_ref, sem, w):
        return pltpu.make_async_copy(
            ix_hbm.at[pl.ds(s * FT + w * W, W)], idx_ref, sem)

    def scat(ox_ref, sem):
        return pltpu.make_async_copy(ones, bins.at[ox_ref], sem)

    def one_pass(p, _):
        cellbase = (2 * p + k) * CHUNK_CELLS
        base_bin = cellbase * N_GENES

        @pl.when(p == 0)
        def _():
            z0 = s * ZSLICE
            for c in range(52):
                pltpu.async_copy(fill, bins.at[pl.ds(z0 + c * W, W)], sem_z)
            pltpu.async_copy(fill.at[pl.ds(0, 512)],
                             bins.at[pl.ds(z0 + 52 * W, 512)], sem_z)
            for c in range(52):
                pltpu.make_async_copy(fill, bins.at[pl.ds(z0 + c * W, W)],
                                      sem_z).wait()
            pltpu.make_async_copy(fill.at[pl.ds(0, 512)],
                                  bins.at[pl.ds(z0 + 52 * W, 512)],
                                  sem_z).wait()
            plsc.subcore_barrier()

        load(idxa, sem_la, 0).start()

        def compute(idx_ref, o1, o2):
            def one_row(r, _):
                for cg in range(8):
                    o = r * 128 + cg * 16
                    v = idx_ref[pl.ds(o, 16)]
                    rel = v - base_bin
                    m = plsc.bitcast(rel, jnp.uint32) < jnp.uint32(CHUNK_BINS)
                    o1[pl.ds(o, 16)] = jnp.where(m, rel, trash_vec)
                    v2 = idx_ref[pl.ds(H + o, 16)]
                    rel2 = v2 - base_bin
                    m2 = plsc.bitcast(rel2, jnp.uint32) < jnp.uint32(CHUNK_BINS)
                    o2[pl.ds(o, 16)] = jnp.where(m2, rel2, trash_vec)
                return 0

            lax.fori_loop(0, H // 128, one_row, 0)

        def turn(cur, nxt, w):
            cur_idx, cur_o1, cur_o2, cur_sl, cur_ss = cur
            nxt_idx, nxt_o1, nxt_o2, nxt_sl, nxt_ss = nxt
            load(cur_idx, cur_sl, w).wait()
            compute(cur_idx, cur_o1, cur_o2)
            pltpu.async_copy(ones, bins.at[cur_o1], cur_ss, add=True)
            pltpu.async_copy(ones, bins.at[cur_o2], cur_ss, add=True)

            @pl.when(w >= 2)
            def _():
                scat(nxt_o1, nxt_ss).wait()
                scat(nxt_o2, nxt_ss).wait()

            @pl.when(w + 1 < NWIN)
            def _():
                load(nxt_idx, nxt_sl, w + 1).start()

        slot_a = (idxa, oxa1, oxa2, sem_la, sem_sa)
        slot_b = (idxb, oxb1, oxb2, sem_lb, sem_sb)
        slot_c = (idxc, oxc1, oxc2, sem_lc, sem_sc)

        def one_triple(t, _):
            turn(slot_a, slot_b, 3 * t)
            turn(slot_b, slot_c, 3 * t + 1)
            turn(slot_c, slot_a, 3 * t + 2)
            return 0

        lax.fori_loop(0, NWIN // 3, one_triple, 0)
        scat(oxb1, sem_sb).wait()
        scat(oxb2, sem_sb).wait()
        scat(oxc1, sem_sc).wait()
        scat(oxc2, sem_sc).wait()
        plsc.subcore_barrier()

        def zfires(cl):
            return tuple(
                pltpu.make_async_copy(
                    fill, bins.at[pl.ds(cl * N_GENES + q * W, W)], sem_z)
                for q in range(4)
            ) + (
                pltpu.make_async_copy(
                    fill.at[pl.ds(0, 1808)],
                    bins.at[pl.ds(cl * N_GENES + 4 * W, 1808)], sem_z),
            )

        def one_cell(r, _):
            cl = s * 11 + r
            cell = cellbase + cl

            @pl.when(jnp.logical_and(cl < CHUNK_CELLS, cell < N_CELLS))
            def _():
                for h in range(2):
                    pltpu.sync_copy(
                        bins.at[pl.ds(cl * N_GENES + h * 5000, 5000)], rowbuf)
                    pltpu.sync_copy(
                        rowbuf,
                        cnt_hbm.at[pl.ds(cell * N_GENES + h * 5000, 5000)])
                for d in zfires(cl):
                    d.start()

            return 0

        lax.fori_loop(0, 11, one_cell, 0)

        def drain_cell(r, _):
            cl = s * 11 + r
            cell = cellbase + cl

            @pl.when(jnp.logical_and(cl < CHUNK_CELLS, cell < N_CELLS))
            def _():
                for d in zfires(cl):
                    d.wait()

            return 0

        lax.fori_loop(0, 11, drain_cell, 0)

        @pl.when(s == 0)
        def _():
            pltpu.sync_copy(fill.at[pl.ds(0, 2128)],
                            bins.at[pl.ds(TRASH_BASE, 2128)])

        plsc.subcore_barrier()
        return 0

    lax.fori_loop(0, PASSES, one_pass, 0)


def _tc_body(cnt_ref, g_ref, ab_ref, out_ref):
    x = g_ref[...] * ab_ref[0, 0] + ab_ref[1, 0]

    def softplus(y):
        return jnp.maximum(y, 0.0) + jnp.log(1.0 + jnp.exp(-jnp.abs(y)))

    log_p = -softplus(-x)
    log_1mp = -softplus(x)
    out_ref[...] = cnt_ref[...] * log_1mp + log_p


def kernel(genecounts, nn_logit_weight, nn_logit_bias, local_cellxgene_ix,
           motif_binsize, n_cells, n_genes):
    f32 = jnp.float32
    a = nn_logit_weight[0, 0] * 10.0 / jnp.asarray(motif_binsize, f32)
    b = nn_logit_bias[0]
    ab = jnp.stack([a, b]).reshape(2, 1).astype(f32)

    ix = jnp.concatenate([
        local_cellxgene_ix.astype(jnp.int32),
        jnp.full((NP - N_FRAG,), PAD_VAL, jnp.int32),
    ])

    counts = pl.kernel(
        _sc_body,
        out_type=jax.ShapeDtypeStruct((N_CELLS * N_GENES,), f32),
        mesh=plsc.VectorSubcoreMesh(
            core_axis_name="c", subcore_axis_name="s",
            num_cores=NC, num_subcores=NS,
        ),
        scratch_types=[
            pltpu.VMEM_SHARED((SP_BINS,), f32),
            pltpu.VMEM((W,), jnp.int32),
            pltpu.VMEM((W,), jnp.int32),
            pltpu.VMEM((W,), jnp.int32),
            pltpu.VMEM((H,), jnp.int32),
            pltpu.VMEM((H,), jnp.int32),
            pltpu.VMEM((H,), jnp.int32),
            pltpu.VMEM((H,), jnp.int32),
            pltpu.VMEM((H,), jnp.int32),
            pltpu.VMEM((H,), jnp.int32),
            pltpu.VMEM((W,), f32),
            pltpu.VMEM((H,), f32),
            pltpu.VMEM((5000,), f32),
            pltpu.SemaphoreType.DMA,
            pltpu.SemaphoreType.DMA,
            pltpu.SemaphoreType.DMA,
            pltpu.SemaphoreType.DMA,
            pltpu.SemaphoreType.DMA,
            pltpu.SemaphoreType.DMA,
            pltpu.SemaphoreType.DMA,
        ],
    )(ix)

    return pl.pallas_call(
        _tc_body,
        out_shape=jax.ShapeDtypeStruct((N_CELLS, N_GENES), f32),
        grid=(N_CELLS // RBLK,),
        in_specs=[
            pl.BlockSpec((RBLK, N_GENES), lambda i: (i, 0)),
            pl.BlockSpec((1, N_GENES), lambda i: (0, 0)),
            pl.BlockSpec(memory_space=pltpu.SMEM),
        ],
        out_specs=pl.BlockSpec((RBLK, N_GENES), lambda i: (i, 0)),
    )(counts.reshape(N_CELLS, N_GENES),
      genecounts.astype(f32).reshape(1, N_GENES), ab)


# speedup vs baseline: 1.1176x; 1.1176x over previous
import jax
import jax.numpy as jnp
from jax import lax
from jax.experimental import pallas as pl
from jax.experimental.pallas import tpu as pltpu
from jax.experimental.pallas import tpu_sc as plsc

N_CELLS = 1024
N_GENES = 10000
N_FRAG = 5_000_000

NC = 2
NS = 16

PASSES = 3
CHUNK_CELLS = 171
CHUNK_BINS = CHUNK_CELLS * N_GENES
ZSLICE = 107_008
SP_BINS = ZSLICE * NS
TRASH_BASE = CHUNK_BINS
PAD_VAL = 10_260_000

W = 4096
NWIN = 78
FT = W * NWIN
NP = FT * NS

RBLK = 128


def _sc_body(ix_hbm, cnt_hbm, bins, idxa, idxb, idxc, fill, rowbuf,
             sem_la, sem_lb, sem_lc, sem_sa, sem_sb, sem_sc, sem_z):
    k = lax.axis_index("c")
    s = lax.axis_index("s")

    lane = lax.broadcasted_iota(jnp.int32, (16,), 0)
    trash_vec = TRASH_BASE + s * 16 + lane

    def set_fill(val):
        def body(r, _):
            fill[pl.ds(r * 16, 16)] = jnp.full((16,), val, jnp.float32)
            return 0

        lax.fori_loop(0, W // 16, body, 0)

    def load(idx_ref, sem, w):
        return pltpu.make_async_copy(
            ix_hbm.at[pl.ds(s * FT + w * W, W)], idx_ref, sem)

    def scat(idx_ref, sem):
        return pltpu.make_async_copy(fill, bins.at[idx_ref], sem)

    def one_pass(p, _):
        cellbase = (2 * p + k) * CHUNK_CELLS
        base_bin = cellbase * N_GENES

        @pl.when(p == 0)
        def _():
            set_fill(0.0)
            z0 = s * ZSLICE
            for c in range(26):
                pltpu.async_copy(fill, bins.at[pl.ds(z0 + c * W, W)], sem_z)
            pltpu.async_copy(fill.at[pl.ds(0, 512)],
                             bins.at[pl.ds(z0 + 26 * W, 512)], sem_z)
            for c in range(26):
                pltpu.make_async_copy(fill, bins.at[pl.ds(z0 + c * W, W)],
                                      sem_z).wait()
            pltpu.make_async_copy(fill.at[pl.ds(0, 512)],
                                  bins.at[pl.ds(z0 + 26 * W, 512)],
                                  sem_z).wait()
            plsc.subcore_barrier()

        set_fill(1.0)

        load(idxa, sem_la, 0).start()

        def compute(idx_ref):
            def one_row(r, _):
                for cg in range(8):
                    o = r * 128 + cg * 16
                    v = idx_ref[pl.ds(o, 16)]
                    rel = v - base_bin
                    m = plsc.bitcast(rel, jnp.uint32) < jnp.uint32(CHUNK_BINS)
                    idx_ref[pl.ds(o, 16)] = jnp.where(m, rel, trash_vec)
                return 0

            lax.fori_loop(0, W // 128, one_row, 0)

        def turn(cur, nxt, w):
            cur_ref, cur_sl, cur_ss = cur
            nxt_ref, nxt_sl, nxt_ss = nxt
            load(cur_ref, cur_sl, w).wait()
            compute(cur_ref)
            pltpu.async_copy(fill, bins.at[cur_ref], cur_ss, add=True)

            @pl.when(w >= 2)
            def _():
                scat(nxt_ref, nxt_ss).wait()

            @pl.when(w + 1 < NWIN)
            def _():
                load(nxt_ref, nxt_sl, w + 1).start()

        slot_a = (idxa, sem_la, sem_sa)
        slot_b = (idxb, sem_lb, sem_sb)
        slot_c = (idxc, sem_lc, sem_sc)

        def one_triple(t, _):
            turn(slot_a, slot_b, 3 * t)
            turn(slot_b, slot_c, 3 * t + 1)
            turn(slot_c, slot_a, 3 * t + 2)
            return 0

        lax.fori_loop(0, NWIN // 3, one_triple, 0)
        scat(idxb, sem_sb).wait()
        scat(idxc, sem_sc).wait()
        plsc.subcore_barrier()

        set_fill(0.0)

        def zfires(cl):
            return (
                pltpu.make_async_copy(
                    fill, bins.at[pl.ds(cl * N_GENES, W)], sem_z),
                pltpu.make_async_copy(
                    fill, bins.at[pl.ds(cl * N_GENES + W, W)], sem_z),
                pltpu.make_async_copy(
                    fill.at[pl.ds(0, 1808)],
                    bins.at[pl.ds(cl * N_GENES + 2 * W, 1808)], sem_z),
            )

        def one_cell(r, _):
            cl = s * 11 + r
            cell = cellbase + cl

            @pl.when(jnp.logical_and(cl < CHUNK_CELLS, cell < N_CELLS))
            def _():
                for h in range(2):
                    pltpu.sync_copy(
                        bins.at[pl.ds(cl * N_GENES + h * 5000, 5000)], rowbuf)
                    pltpu.sync_copy(
                        rowbuf,
                        cnt_hbm.at[pl.ds(cell * N_GENES + h * 5000, 5000)])
                for d in zfires(cl):
                    d.start()

            return 0

        lax.fori_loop(0, 11, one_cell, 0)

        def drain_cell(r, _):
            cl = s * 11 + r
            cell = cellbase + cl

            @pl.when(jnp.logical_and(cl < CHUNK_CELLS, cell < N_CELLS))
            def _():
                for d in zfires(cl):
                    d.wait()

            return 0

        lax.fori_loop(0, 11, drain_cell, 0)

        @pl.when(s == 0)
        def _():
            pltpu.sync_copy(fill.at[pl.ds(0, 2128)],
                            bins.at[pl.ds(TRASH_BASE, 2128)])

        plsc.subcore_barrier()
        return 0

    lax.fori_loop(0, PASSES, one_pass, 0)


def _tc_body(cnt_ref, g_ref, ab_ref, out_ref):
    x = g_ref[...] * ab_ref[0, 0] + ab_ref[1, 0]

    def softplus(y):
        return jnp.maximum(y, 0.0) + jnp.log(1.0 + jnp.exp(-jnp.abs(y)))

    log_p = -softplus(-x)
    log_1mp = -softplus(x)
    out_ref[...] = cnt_ref[...] * log_1mp + log_p


def kernel(genecounts, nn_logit_weight, nn_logit_bias, local_cellxgene_ix,
           motif_binsize, n_cells, n_genes):
    f32 = jnp.float32
    a = nn_logit_weight[0, 0] * 10.0 / jnp.asarray(motif_binsize, f32)
    b = nn_logit_bias[0]
    ab = jnp.stack([a, b]).reshape(2, 1).astype(f32)

    ix = jnp.concatenate([
        local_cellxgene_ix.astype(jnp.int32),
        jnp.full((NP - N_FRAG,), PAD_VAL, jnp.int32),
    ])

    counts = pl.kernel(
        _sc_body,
        out_type=jax.ShapeDtypeStruct((N_CELLS * N_GENES,), f32),
        mesh=plsc.VectorSubcoreMesh(
            core_axis_name="c", subcore_axis_name="s",
            num_cores=NC, num_subcores=NS,
        ),
        scratch_types=[
            pltpu.VMEM_SHARED((SP_BINS,), f32),
            pltpu.VMEM((W,), jnp.int32),
            pltpu.VMEM((W,), jnp.int32),
            pltpu.VMEM((W,), jnp.int32),
            pltpu.VMEM((W,), f32),
            pltpu.VMEM((5000,), f32),
            pltpu.SemaphoreType.DMA,
            pltpu.SemaphoreType.DMA,
            pltpu.SemaphoreType.DMA,
            pltpu.SemaphoreType.DMA,
            pltpu.SemaphoreType.DMA,
            pltpu.SemaphoreType.DMA,
            pltpu.SemaphoreType.DMA,
        ],
    )(ix)

    return pl.pallas_call(
        _tc_body,
        out_shape=jax.ShapeDtypeStruct((N_CELLS, N_GENES), f32),
        grid=(N_CELLS // RBLK,),
        in_specs=[
            pl.BlockSpec((RBLK, N_GENES), lambda i: (i, 0)),
            pl.BlockSpec((1, N_GENES), lambda i: (0, 0)),
            pl.BlockSpec(memory_space=pltpu.SMEM),
        ],
        out_specs=pl.BlockSpec((RBLK, N_GENES), lambda i: (i, 0)),
    )(counts.reshape(N_CELLS, N_GENES),
      genecounts.astype(f32).reshape(1, N_GENES), ab)
